# baseline (speedup 1.0000x reference)
import functools

import jax
import jax.numpy as jnp
from jax import lax
from jax.experimental import pallas as pl
from jax.experimental.pallas import tpu as pltpu
from jax.experimental.pallas import tpu_sc as plsc

N = 10000
E = 320000
D = 128
NP = 10240
NH = NP // 2
NSC = 2
NTILE = 16
NW = NSC * NTILE
WIN = 128
WPT = 80
EP = NW * WPT * WIN
EPT = EP // NW
LHALF = EPT + WIN
LWPT = LHALF // WIN
LTILE = 2 * LHALF
RPT = NP // NTILE
HRPT = NH // NTILE

_mesh = plsc.VectorSubcoreMesh(
    core_axis_name="c", subcore_axis_name="s", num_cores=NSC,
    num_subcores=NTILE)


@functools.partial(
    pl.kernel,
    out_type=[jax.ShapeDtypeStruct((NSC, NP), jnp.float32),
              jax.ShapeDtypeStruct((NSC, NTILE, LTILE), jnp.int32),
              jax.ShapeDtypeStruct((NSC, NTILE, LTILE), jnp.int32)],
    mesh=_mesh,
    scratch_types=[
        pltpu.VMEM((WPT, WIN), jnp.int32),
        pltpu.VMEM((WPT, WIN), jnp.int32),
        pltpu.VMEM((WPT, WIN), jnp.int32),
        pltpu.VMEM((WPT, WIN), jnp.int32),
        pltpu.VMEM((WIN,), jnp.float32),
        pltpu.VMEM_SHARED((NP,), jnp.float32),
        pltpu.VMEM_SHARED((NTILE * LTILE,), jnp.int32),
        pltpu.VMEM_SHARED((NTILE * LTILE,), jnp.int32),
    ],
)
def _sc_prep(dstw_hbm, srcw_hbm, ldw_hbm, posw_hbm, tsrc_hbm, tld_hbm,
             zeros_hbm, hist_hbm, lsrc_hbm, lld_hbm,
             idx_v, srcv, ldv, posv, ones_v, hist_sh, spm_src, spm_ld):
    cid = lax.axis_index("c")
    sid = lax.axis_index("s")
    wid = cid * NTILE + sid
    pltpu.sync_copy(dstw_hbm.at[wid], idx_v)
    pltpu.sync_copy(srcw_hbm.at[wid], srcv)
    pltpu.sync_copy(ldw_hbm.at[wid], ldv)
    pltpu.sync_copy(posw_hbm.at[wid], posv)
    pltpu.sync_copy(zeros_hbm.at[pl.ds(sid * RPT, RPT)],
                    hist_sh.at[pl.ds(sid * RPT, RPT)])
    pltpu.sync_copy(tsrc_hbm, spm_src.at[pl.ds(sid * LTILE, LTILE)])
    pltpu.sync_copy(tld_hbm, spm_ld.at[pl.ds(sid * LTILE, LTILE)])
    for i in range(WIN // 16):
        ones_v[pl.ds(i * 16, 16)] = jnp.ones((16,), jnp.float32)
    plsc.subcore_barrier()

    def body(w, carry):
        pltpu.sync_copy(ones_v, hist_sh.at[idx_v.at[w]], add=True)
        pltpu.sync_copy(srcv.at[w], spm_src.at[posv.at[w]])
        pltpu.sync_copy(ldv.at[w], spm_ld.at[posv.at[w]])
        return carry

    lax.fori_loop(0, WPT, body, 0)
    plsc.subcore_barrier()
    pltpu.sync_copy(hist_sh.at[pl.ds(sid * RPT, RPT)],
                    hist_hbm.at[cid, pl.ds(sid * RPT, RPT)])
    pltpu.sync_copy(spm_src.at[pl.ds(sid * LTILE, LTILE)],
                    lsrc_hbm.at[cid, sid])
    pltpu.sync_copy(spm_ld.at[pl.ds(sid * LTILE, LTILE)],
                    lld_hbm.at[cid, sid])


@functools.partial(
    pl.kernel,
    out_type=jax.ShapeDtypeStruct((NSC, NP, D), jnp.float32),
    mesh=_mesh,
    scratch_types=[
        pltpu.VMEM((LWPT, WIN), jnp.int32),
        pltpu.VMEM((LWPT, WIN), jnp.int32),
        pltpu.VMEM((16,), jnp.int32),
        pltpu.VMEM((WIN, D), jnp.float32),
        pltpu.VMEM((WIN, D), jnp.float32),
        pltpu.VMEM_SHARED((NH, D), jnp.float32),
        pltpu.SemaphoreType.DMA,
        pltpu.SemaphoreType.DMA,
        pltpu.SemaphoreType.DMA,
        pltpu.SemaphoreType.DMA,
    ],
)
def _sc_scatter(gl_hbm, ll_hbm, cnt_hbm, y_hbm, out_hbm,
                gf, lf, cntv, buf0, buf1, acc, sem0, sem1, ssm0, ssm1):
    cid = lax.axis_index("c")
    sid = lax.axis_index("s")
    wid = cid * NTILE + sid
    pltpu.sync_copy(cnt_hbm.at[wid], cntv)

    bufs = (buf0, buf1)
    sems = (sem0, sem1)
    ssems = (ssm0, ssm1)

    for half in (0, 1):
        base = half * NH
        pltpu.sync_copy(gl_hbm.at[wid, half], gf)
        pltpu.sync_copy(ll_hbm.at[wid, half], lf)
        cw = cntv[pl.ds(0, 16)][half]
        pltpu.sync_copy(y_hbm.at[pl.ds(base + sid * HRPT, HRPT)],
                        acc.at[pl.ds(sid * HRPT, HRPT)])
        plsc.subcore_barrier()

        @pl.when(cw > 0)
        def _():
            pltpu.async_copy(y_hbm.at[gf.at[0]], buf0, sem0)

        @pl.when(cw > WIN)
        def _():
            pltpu.async_copy(y_hbm.at[gf.at[1]], buf1, sem1)

        def body(i, carry):
            for b in range(2):
                w = 2 * i + b

                @pl.when(w * WIN < cw)
                def _():
                    pltpu.make_async_copy(y_hbm.at[gf.at[w]], bufs[b],
                                          sems[b]).wait()
                    pltpu.async_copy(bufs[b], acc.at[lf.at[w]], ssems[b],
                                     add=True)
            for b in range(2):
                w = 2 * i + b

                @pl.when(w * WIN < cw)
                def _():
                    pltpu.make_async_copy(bufs[b], acc.at[lf.at[0]],
                                          ssems[b]).wait()

                    @pl.when((w + 2) * WIN < cw)
                    def _():
                        pltpu.async_copy(y_hbm.at[gf.at[w + 2]], bufs[b],
                                         sems[b])
            return carry

        lax.fori_loop(0, (LWPT + 1) // 2, body, 0)
        plsc.subcore_barrier()
        pltpu.sync_copy(acc.at[pl.ds(sid * HRPT, HRPT)],
                        out_hbm.at[cid, pl.ds(base + sid * HRPT, HRPT)])


R = 1280
_HI = lax.Precision.HIGHEST


def _tc_pos_body(d_ref, pos_out, cnt_out):
    d = d_ref[...]
    m0 = (d < NH).astype(jnp.float32)
    m1 = 1.0 - m0
    rc = lax.broadcasted_iota(jnp.int32, (WIN, WIN), 0)
    cc = lax.broadcasted_iota(jnp.int32, (WIN, WIN), 1)
    u = (rc < cc).astype(jnp.float32)
    rr = lax.broadcasted_iota(jnp.int32, (WPT, WPT), 0)
    cr = lax.broadcasted_iota(jnp.int32, (WPT, WPT), 1)
    t = (cr < rr).astype(jnp.float32)

    def prefix(m):
        wexc = jnp.dot(m, u, precision=_HI,
                       preferred_element_type=jnp.float32)
        rows = jnp.sum(m, axis=1, keepdims=True)
        rexc = jnp.dot(t, rows, precision=_HI,
                       preferred_element_type=jnp.float32)
        return wexc + rexc, jnp.sum(rows)

    p0, c0 = prefix(m0)
    p1, c1 = prefix(m1)
    pos = jnp.where(d < NH, p0, LHALF + p1).astype(jnp.int32)
    pos_out[...] = pos + (pl.program_id(0) % NTILE) * LTILE
    cnt = jnp.concatenate(
        [c0.reshape(1, 1), c1.reshape(1, 1), jnp.zeros((1, 126))], axis=1)
    cnt_out[...] = (cnt + jnp.zeros((8, 128))).astype(jnp.int32)


_tc_pos = pl.pallas_call(
    _tc_pos_body,
    grid=(NW,),
    in_specs=[pl.BlockSpec((WPT, WIN), lambda i: (i, 0))],
    out_specs=[pl.BlockSpec((WPT, WIN), lambda i: (i, 0)),
               pl.BlockSpec((8, 128), lambda i: (i, 0))],
    out_shape=[jax.ShapeDtypeStruct((NW * WPT, WIN), jnp.int32),
               jax.ShapeDtypeStruct((NW * 8, 128), jnp.int32)],
)


def _tc_first_body(h0, h1, x, w, y_out, dinv_out):
    deg = h0[...] + h1[...] + 1.0
    row = pl.program_id(0) * R + lax.broadcasted_iota(jnp.int32, (R, 1), 0)
    dinv = jnp.where(row < N, lax.rsqrt(deg), 0.0)
    y_out[...] = jnp.dot(x[...], w[...], precision=_HI,
                         preferred_element_type=jnp.float32) * dinv
    dinv_out[...] = dinv


def _tc_mid_body(p0, p1, y, dinv, b, w, yn_out):
    h = jnp.maximum(dinv[...] * (p0[...] + p1[...] - y[...]) + b[...], 0.0)
    yn_out[...] = jnp.dot(h, w[...], precision=_HI,
                          preferred_element_type=jnp.float32) * dinv[...]


def _tc_last_body(p0, p1, y, dinv, b, out):
    h = dinv[...] * (p0[...] + p1[...] - y[...]) + b[...]
    m = jnp.max(h, axis=-1, keepdims=True)
    s = jnp.sum(jnp.exp(h - m), axis=-1, keepdims=True)
    out[...] = h - m - jnp.log(s)


_col = pl.BlockSpec((R, 1), lambda i: (i, 0))
_rowblk = pl.BlockSpec((R, D), lambda i: (i, 0))
_wspec = pl.BlockSpec((D, D), lambda i: (0, 0))
_bspec = pl.BlockSpec((1, D), lambda i: (0, 0))
_grid = NP // R

_tc_first = pl.pallas_call(
    _tc_first_body,
    grid=(_grid,),
    in_specs=[_col, _col, _rowblk, _wspec],
    out_specs=[_rowblk, _col],
    out_shape=[jax.ShapeDtypeStruct((NP, D), jnp.float32),
               jax.ShapeDtypeStruct((NP, 1), jnp.float32)],
)

_tc_mid = pl.pallas_call(
    _tc_mid_body,
    grid=(_grid,),
    in_specs=[_rowblk, _rowblk, _rowblk, _col, _bspec, _wspec],
    out_specs=_rowblk,
    out_shape=jax.ShapeDtypeStruct((NP, D), jnp.float32),
)

_tc_last = pl.pallas_call(
    _tc_last_body,
    grid=(_grid,),
    in_specs=[_rowblk, _rowblk, _rowblk, _col, _bspec],
    out_specs=_rowblk,
    out_shape=jax.ShapeDtypeStruct((NP, D), jnp.float32),
)


def kernel(x, edge_index, batch, W1, b1, W2, b2, W3, b3):
    src = edge_index[0].astype(jnp.int32)
    dst = edge_index[1].astype(jnp.int32)
    npad = EP - E
    ar = jnp.arange(npad, dtype=jnp.int32)
    srcf = jnp.concatenate([src, N + (ar % 128)])
    dstf = jnp.concatenate([dst, N + (ar % 8)])
    ldf = jnp.where(dstf >= NH, dstf - NH, dstf)
    dstw = dstf.reshape(NW, WPT, WIN)
    srcw = srcf.reshape(NW, WPT, WIN)
    ldw = ldf.reshape(NW, WPT, WIN)
    xp = jnp.zeros((NP, D), jnp.float32).at[:N].set(x)
    zeros1d = jnp.zeros((NP,), jnp.float32)
    tsrc = N + (jnp.arange(LTILE, dtype=jnp.int32) % 128)
    tld = jnp.zeros((LTILE,), jnp.int32)

    posb, cntb = _tc_pos(dstf.reshape(NW * WPT, WIN))
    posw = posb.reshape(NW, WPT, WIN)
    cnt = cntb.reshape(NW, 8, 128)[:, 0, :16]
    hist, lsrc, lld = _sc_prep(dstw, srcw, ldw, posw, tsrc, tld, zeros1d)
    gl = lsrc.reshape(NW, 2, LWPT, WIN)
    ll = lld.reshape(NW, 2, LWPT, WIN)

    y, dinv = _tc_first(hist[0][:, None], hist[1][:, None], xp, W1)
    p = _sc_scatter(gl, ll, cnt, y)
    y = _tc_mid(p[0], p[1], y, dinv, b1.reshape(1, D), W2)
    p = _sc_scatter(gl, ll, cnt, y)
    y = _tc_mid(p[0], p[1], y, dinv, b2.reshape(1, D), W3)
    p = _sc_scatter(gl, ll, cnt, y)
    out = _tc_last(p[0], p[1], y, dinv, b3.reshape(1, D))
    return out[:N]


# speedup vs baseline: 1.2691x; 1.2691x over previous
import functools

import jax
import jax.numpy as jnp
from jax import lax
from jax.experimental import pallas as pl
from jax.experimental.pallas import tpu as pltpu
from jax.experimental.pallas import tpu_sc as plsc

N = 10000
E = 320000
D = 128
NP = 10240
NH = NP // 2
NSC = 2
NTILE = 16
NW = NSC * NTILE
WIN = 128
WPT = 80
EP = NW * WPT * WIN
EPT = EP // NW
LHALF = EPT + WIN
LWPT = LHALF // WIN
LTILE = 2 * LHALF
RPT = NP // NTILE
HRPT = NH // NTILE

_mesh = plsc.VectorSubcoreMesh(
    core_axis_name="c", subcore_axis_name="s", num_cores=NSC,
    num_subcores=NTILE)


@functools.partial(
    pl.kernel,
    out_type=[jax.ShapeDtypeStruct((NSC, NP), jnp.float32),
              jax.ShapeDtypeStruct((NSC, NTILE, LTILE), jnp.int32),
              jax.ShapeDtypeStruct((NSC, NTILE, LTILE), jnp.int32)],
    mesh=_mesh,
    scratch_types=[
        pltpu.VMEM((WPT, WIN), jnp.int32),
        pltpu.VMEM((WPT, WIN), jnp.int32),
        pltpu.VMEM((WPT, WIN), jnp.int32),
        pltpu.VMEM((WPT, WIN), jnp.int32),
        pltpu.VMEM((WIN,), jnp.float32),
        pltpu.VMEM_SHARED((NP,), jnp.float32),
        pltpu.VMEM_SHARED((NTILE * LTILE,), jnp.int32),
        pltpu.VMEM_SHARED((NTILE * LTILE,), jnp.int32),
    ],
)
def _sc_prep(dstw_hbm, srcw_hbm, ldw_hbm, posw_hbm, tsrc_hbm, tld_hbm,
             zeros_hbm, hist_hbm, lsrc_hbm, lld_hbm,
             idx_v, srcv, ldv, posv, ones_v, hist_sh, spm_src, spm_ld):
    cid = lax.axis_index("c")
    sid = lax.axis_index("s")
    wid = cid * NTILE + sid
    pltpu.sync_copy(dstw_hbm.at[wid], idx_v)
    pltpu.sync_copy(srcw_hbm.at[wid], srcv)
    pltpu.sync_copy(ldw_hbm.at[wid], ldv)
    pltpu.sync_copy(posw_hbm.at[wid], posv)
    pltpu.sync_copy(zeros_hbm.at[pl.ds(sid * RPT, RPT)],
                    hist_sh.at[pl.ds(sid * RPT, RPT)])
    pltpu.sync_copy(tsrc_hbm, spm_src.at[pl.ds(sid * LTILE, LTILE)])
    pltpu.sync_copy(tld_hbm, spm_ld.at[pl.ds(sid * LTILE, LTILE)])
    for i in range(WIN // 16):
        ones_v[pl.ds(i * 16, 16)] = jnp.ones((16,), jnp.float32)
    plsc.subcore_barrier()

    def body(w, carry):
        pltpu.sync_copy(ones_v, hist_sh.at[idx_v.at[w]], add=True)
        pltpu.sync_copy(srcv.at[w], spm_src.at[posv.at[w]])
        pltpu.sync_copy(ldv.at[w], spm_ld.at[posv.at[w]])
        return carry

    lax.fori_loop(0, WPT, body, 0)
    plsc.subcore_barrier()
    pltpu.sync_copy(hist_sh.at[pl.ds(sid * RPT, RPT)],
                    hist_hbm.at[cid, pl.ds(sid * RPT, RPT)])
    pltpu.sync_copy(spm_src.at[pl.ds(sid * LTILE, LTILE)],
                    lsrc_hbm.at[cid, sid])
    pltpu.sync_copy(spm_ld.at[pl.ds(sid * LTILE, LTILE)],
                    lld_hbm.at[cid, sid])


@functools.partial(
    pl.kernel,
    out_type=jax.ShapeDtypeStruct((NSC, NP, D), jnp.float32),
    mesh=_mesh,
    scratch_types=[
        pltpu.VMEM((LWPT, WIN), jnp.int32),
        pltpu.VMEM((LWPT, WIN), jnp.int32),
        pltpu.VMEM((16,), jnp.int32),
        pltpu.VMEM((WIN, D), jnp.float32),
        pltpu.VMEM((WIN, D), jnp.float32),
        pltpu.VMEM_SHARED((NH, D), jnp.float32),
        pltpu.SemaphoreType.DMA,
        pltpu.SemaphoreType.DMA,
    ],
)
def _sc_scatter(gl_hbm, ll_hbm, cnt_hbm, y_hbm, out_hbm,
                gf, lf, cntv, buf0, buf1, acc, sem0, sem1):
    cid = lax.axis_index("c")
    sid = lax.axis_index("s")
    wid = cid * NTILE + sid
    pltpu.sync_copy(cnt_hbm.at[wid], cntv)

    bufs = (buf0, buf1)
    sems = (sem0, sem1)

    for half in (0, 1):
        base = half * NH
        pltpu.sync_copy(gl_hbm.at[wid, half], gf)
        pltpu.sync_copy(ll_hbm.at[wid, half], lf)
        cw = cntv[pl.ds(0, 16)][half]
        pltpu.sync_copy(y_hbm.at[pl.ds(base + sid * HRPT, HRPT)],
                        acc.at[pl.ds(sid * HRPT, HRPT)])
        plsc.subcore_barrier()

        @pl.when(cw > 0)
        def _():
            pltpu.async_copy(y_hbm.at[gf.at[0]], buf0, sem0)

        @pl.when(cw > WIN)
        def _():
            pltpu.async_copy(y_hbm.at[gf.at[1]], buf1, sem1)

        def body(i, carry):
            for b in range(2):
                w = 2 * i + b

                @pl.when(w * WIN < cw)
                def _():
                    pltpu.make_async_copy(y_hbm.at[gf.at[w]], bufs[b],
                                          sems[b]).wait()
                    pltpu.sync_copy(bufs[b], acc.at[lf.at[w]], add=True)

                    @pl.when((w + 2) * WIN < cw)
                    def _():
                        pltpu.async_copy(y_hbm.at[gf.at[w + 2]], bufs[b],
                                         sems[b])
            return carry

        lax.fori_loop(0, (LWPT + 1) // 2, body, 0)
        plsc.subcore_barrier()
        pltpu.sync_copy(acc.at[pl.ds(sid * HRPT, HRPT)],
                        out_hbm.at[cid, pl.ds(base + sid * HRPT, HRPT)])


R = 1280
_HI = lax.Precision.HIGHEST


def _tc_pos_body(d_ref, pos_out, cnt_out):
    d = d_ref[...]
    m0 = (d < NH).astype(jnp.float32)
    m1 = 1.0 - m0
    rc = lax.broadcasted_iota(jnp.int32, (WIN, WIN), 0)
    cc = lax.broadcasted_iota(jnp.int32, (WIN, WIN), 1)
    u = (rc < cc).astype(jnp.float32)
    rr = lax.broadcasted_iota(jnp.int32, (WPT, WPT), 0)
    cr = lax.broadcasted_iota(jnp.int32, (WPT, WPT), 1)
    t = (cr < rr).astype(jnp.float32)

    def prefix(m):
        wexc = jnp.dot(m, u, precision=_HI,
                       preferred_element_type=jnp.float32)
        rows = jnp.sum(m, axis=1, keepdims=True)
        rexc = jnp.dot(t, rows, precision=_HI,
                       preferred_element_type=jnp.float32)
        return wexc + rexc, jnp.sum(rows)

    p0, c0 = prefix(m0)
    p1, c1 = prefix(m1)
    pos = jnp.where(d < NH, p0, LHALF + p1).astype(jnp.int32)
    pos_out[...] = pos + (pl.program_id(0) % NTILE) * LTILE
    cnt = jnp.concatenate(
        [c0.reshape(1, 1), c1.reshape(1, 1), jnp.zeros((1, 126))], axis=1)
    cnt_out[...] = (cnt + jnp.zeros((8, 128))).astype(jnp.int32)


_tc_pos = pl.pallas_call(
    _tc_pos_body,
    grid=(NW,),
    in_specs=[pl.BlockSpec((WPT, WIN), lambda i: (i, 0))],
    out_specs=[pl.BlockSpec((WPT, WIN), lambda i: (i, 0)),
               pl.BlockSpec((8, 128), lambda i: (i, 0))],
    out_shape=[jax.ShapeDtypeStruct((NW * WPT, WIN), jnp.int32),
               jax.ShapeDtypeStruct((NW * 8, 128), jnp.int32)],
)


def _tc_first_body(h0, h1, x, w, y_out, dinv_out):
    deg = h0[...] + h1[...] + 1.0
    row = pl.program_id(0) * R + lax.broadcasted_iota(jnp.int32, (R, 1), 0)
    dinv = jnp.where(row < N, lax.rsqrt(deg), 0.0)
    y_out[...] = jnp.dot(x[...], w[...], precision=_HI,
                         preferred_element_type=jnp.float32) * dinv
    dinv_out[...] = dinv


def _tc_mid_body(p0, p1, y, dinv, b, w, yn_out):
    h = jnp.maximum(dinv[...] * (p0[...] + p1[...] - y[...]) + b[...], 0.0)
    yn_out[...] = jnp.dot(h, w[...], precision=_HI,
                          preferred_element_type=jnp.float32) * dinv[...]


def _tc_last_body(p0, p1, y, dinv, b, out):
    h = dinv[...] * (p0[...] + p1[...] - y[...]) + b[...]
    m = jnp.max(h, axis=-1, keepdims=True)
    s = jnp.sum(jnp.exp(h - m), axis=-1, keepdims=True)
    out[...] = h - m - jnp.log(s)


_col = pl.BlockSpec((R, 1), lambda i: (i, 0))
_rowblk = pl.BlockSpec((R, D), lambda i: (i, 0))
_wspec = pl.BlockSpec((D, D), lambda i: (0, 0))
_bspec = pl.BlockSpec((1, D), lambda i: (0, 0))
_grid = NP // R

_tc_first = pl.pallas_call(
    _tc_first_body,
    grid=(_grid,),
    in_specs=[_col, _col, _rowblk, _wspec],
    out_specs=[_rowblk, _col],
    out_shape=[jax.ShapeDtypeStruct((NP, D), jnp.float32),
               jax.ShapeDtypeStruct((NP, 1), jnp.float32)],
)

_tc_mid = pl.pallas_call(
    _tc_mid_body,
    grid=(_grid,),
    in_specs=[_rowblk, _rowblk, _rowblk, _col, _bspec, _wspec],
    out_specs=_rowblk,
    out_shape=jax.ShapeDtypeStruct((NP, D), jnp.float32),
)

_tc_last = pl.pallas_call(
    _tc_last_body,
    grid=(_grid,),
    in_specs=[_rowblk, _rowblk, _rowblk, _col, _bspec],
    out_specs=_rowblk,
    out_shape=jax.ShapeDtypeStruct((NP, D), jnp.float32),
)


def kernel(x, edge_index, batch, W1, b1, W2, b2, W3, b3):
    src = edge_index[0].astype(jnp.int32)
    dst = edge_index[1].astype(jnp.int32)
    npad = EP - E
    ar = jnp.arange(npad, dtype=jnp.int32)
    srcf = jnp.concatenate([src, N + (ar % 128)])
    dstf = jnp.concatenate([dst, N + (ar % 8)])
    ldf = jnp.where(dstf >= NH, dstf - NH, dstf)
    dstw = dstf.reshape(NW, WPT, WIN)
    srcw = srcf.reshape(NW, WPT, WIN)
    ldw = ldf.reshape(NW, WPT, WIN)
    xp = jnp.zeros((NP, D), jnp.float32).at[:N].set(x)
    zeros1d = jnp.zeros((NP,), jnp.float32)
    tsrc = N + (jnp.arange(LTILE, dtype=jnp.int32) % 128)
    tld = jnp.zeros((LTILE,), jnp.int32)

    posb, cntb = _tc_pos(dstf.reshape(NW * WPT, WIN))
    posw = posb.reshape(NW, WPT, WIN)
    cnt = cntb.reshape(NW, 8, 128)[:, 0, :16]
    hist, lsrc, lld = _sc_prep(dstw, srcw, ldw, posw, tsrc, tld, zeros1d)
    gl = lsrc.reshape(NW, 2, LWPT, WIN)
    ll = lld.reshape(NW, 2, LWPT, WIN)

    y, dinv = _tc_first(hist[0][:, None], hist[1][:, None], xp, W1)
    p = _sc_scatter(gl, ll, cnt, y)
    y = _tc_mid(p[0], p[1], y, dinv, b1.reshape(1, D), W2)
    p = _sc_scatter(gl, ll, cnt, y)
    y = _tc_mid(p[0], p[1], y, dinv, b2.reshape(1, D), W3)
    p = _sc_scatter(gl, ll, cnt, y)
    out = _tc_last(p[0], p[1], y, dinv, b3.reshape(1, D))
    return out[:N]


# speedup vs baseline: 1.3871x; 1.0929x over previous
import functools

import jax
import jax.numpy as jnp
from jax import lax
from jax.experimental import pallas as pl
from jax.experimental.pallas import tpu as pltpu
from jax.experimental.pallas import tpu_sc as plsc

N = 10000
E = 320000
D = 128
NP = 10240
NH = NP // 2
NSC = 2
NTILE = 16
NW = NSC * NTILE
WIN = 128
WPT = 80
EP = NW * WPT * WIN
EPT = EP // NW
LHALF = EPT + WIN
LWPT = LHALF // WIN
LTILE = 2 * LHALF
RPT = NP // NTILE
HRPT = NH // NTILE

_mesh = plsc.VectorSubcoreMesh(
    core_axis_name="c", subcore_axis_name="s", num_cores=NSC,
    num_subcores=NTILE)


@functools.partial(
    pl.kernel,
    out_type=[jax.ShapeDtypeStruct((NSC, NP), jnp.float32),
              jax.ShapeDtypeStruct((NSC, NTILE, LTILE), jnp.int32),
              jax.ShapeDtypeStruct((NSC, NTILE, LTILE), jnp.int32)],
    mesh=_mesh,
    scratch_types=[
        pltpu.VMEM((WPT, WIN), jnp.int32),
        pltpu.VMEM((WPT, WIN), jnp.int32),
        pltpu.VMEM((WPT, WIN), jnp.int32),
        pltpu.VMEM((WPT, WIN), jnp.int32),
        pltpu.VMEM((WIN,), jnp.float32),
        pltpu.VMEM_SHARED((NP,), jnp.float32),
        pltpu.VMEM_SHARED((NTILE * LTILE,), jnp.int32),
        pltpu.VMEM_SHARED((NTILE * LTILE,), jnp.int32),
    ],
)
def _sc_prep(dstw_hbm, srcw_hbm, ldw_hbm, posw_hbm, tsrc_hbm, tld_hbm,
             zeros_hbm, hist_hbm, lsrc_hbm, lld_hbm,
             idx_v, srcv, ldv, posv, ones_v, hist_sh, spm_src, spm_ld):
    cid = lax.axis_index("c")
    sid = lax.axis_index("s")
    wid = cid * NTILE + sid
    pltpu.sync_copy(dstw_hbm.at[wid], idx_v)
    pltpu.sync_copy(srcw_hbm.at[wid], srcv)
    pltpu.sync_copy(ldw_hbm.at[wid], ldv)
    pltpu.sync_copy(posw_hbm.at[wid], posv)
    pltpu.sync_copy(zeros_hbm.at[pl.ds(sid * RPT, RPT)],
                    hist_sh.at[pl.ds(sid * RPT, RPT)])
    pltpu.sync_copy(tsrc_hbm, spm_src.at[pl.ds(sid * LTILE, LTILE)])
    pltpu.sync_copy(tld_hbm, spm_ld.at[pl.ds(sid * LTILE, LTILE)])
    for i in range(WIN // 16):
        ones_v[pl.ds(i * 16, 16)] = jnp.ones((16,), jnp.float32)
    plsc.subcore_barrier()

    def body(w, carry):
        pltpu.sync_copy(ones_v, hist_sh.at[idx_v.at[w]], add=True)
        pltpu.sync_copy(srcv.at[w], spm_src.at[posv.at[w]])
        pltpu.sync_copy(ldv.at[w], spm_ld.at[posv.at[w]])
        return carry

    lax.fori_loop(0, WPT, body, 0)
    plsc.subcore_barrier()
    pltpu.sync_copy(hist_sh.at[pl.ds(sid * RPT, RPT)],
                    hist_hbm.at[cid, pl.ds(sid * RPT, RPT)])
    pltpu.sync_copy(spm_src.at[pl.ds(sid * LTILE, LTILE)],
                    lsrc_hbm.at[cid, sid])
    pltpu.sync_copy(spm_ld.at[pl.ds(sid * LTILE, LTILE)],
                    lld_hbm.at[cid, sid])


@functools.partial(
    pl.kernel,
    out_type=jax.ShapeDtypeStruct((NSC, NP, D), jnp.float32),
    mesh=_mesh,
    scratch_types=[
        pltpu.VMEM((LWPT, WIN), jnp.int32),
        pltpu.VMEM((LWPT, WIN), jnp.int32),
        pltpu.VMEM((16,), jnp.int32),
        pltpu.VMEM((WIN, D), jnp.float32),
        pltpu.VMEM((WIN, D), jnp.float32),
        pltpu.VMEM((WIN, D), jnp.float32),
        pltpu.VMEM_SHARED((NH, D), jnp.float32),
        pltpu.SemaphoreType.DMA,
        pltpu.SemaphoreType.DMA,
        pltpu.SemaphoreType.DMA,
    ],
)
def _sc_scatter(gl_hbm, ll_hbm, cnt_hbm, y_hbm, out_hbm,
                gf, lf, cntv, buf0, buf1, buf2, acc, sem0, sem1, sem2):
    cid = lax.axis_index("c")
    sid = lax.axis_index("s")
    wid = cid * NTILE + sid
    pltpu.sync_copy(cnt_hbm.at[wid], cntv)

    bufs = (buf0, buf1, buf2)
    sems = (sem0, sem1, sem2)

    for half in (0, 1):
        base = half * NH
        pltpu.sync_copy(gl_hbm.at[wid, half], gf)
        pltpu.sync_copy(ll_hbm.at[wid, half], lf)
        cw = cntv[pl.ds(0, 16)][half]
        pltpu.sync_copy(y_hbm.at[pl.ds(base + sid * HRPT, HRPT)],
                        acc.at[pl.ds(sid * HRPT, HRPT)])
        plsc.subcore_barrier()

        @pl.when(cw > 0)
        def _():
            pltpu.async_copy(y_hbm.at[gf.at[0]], buf0, sem0)

        @pl.when(cw > WIN)
        def _():
            pltpu.async_copy(y_hbm.at[gf.at[1]], buf1, sem1)

        @pl.when(cw > 2 * WIN)
        def _():
            pltpu.async_copy(y_hbm.at[gf.at[2]], buf2, sem2)

        def body(i, carry):
            for b in range(3):
                w = 3 * i + b

                @pl.when(w * WIN < cw)
                def _():
                    pltpu.make_async_copy(y_hbm.at[gf.at[w]], bufs[b],
                                          sems[b]).wait()
                    pltpu.sync_copy(bufs[b], acc.at[lf.at[w]], add=True)

                    @pl.when((w + 3) * WIN < cw)
                    def _():
                        pltpu.async_copy(y_hbm.at[gf.at[w + 3]], bufs[b],
                                         sems[b])
            return carry

        lax.fori_loop(0, (LWPT + 2) // 3, body, 0)
        plsc.subcore_barrier()
        pltpu.sync_copy(acc.at[pl.ds(sid * HRPT, HRPT)],
                        out_hbm.at[cid, pl.ds(base + sid * HRPT, HRPT)])


R = 1280
_HI = lax.Precision.HIGHEST


def _tc_pos_body(d_ref, pos_out, cnt_out):
    d = d_ref[...]
    m0 = (d < NH).astype(jnp.float32)
    m1 = 1.0 - m0
    rc = lax.broadcasted_iota(jnp.int32, (WIN, WIN), 0)
    cc = lax.broadcasted_iota(jnp.int32, (WIN, WIN), 1)
    u = (rc < cc).astype(jnp.float32)
    rr = lax.broadcasted_iota(jnp.int32, (WPT, WPT), 0)
    cr = lax.broadcasted_iota(jnp.int32, (WPT, WPT), 1)
    t = (cr < rr).astype(jnp.float32)

    def prefix(m):
        wexc = jnp.dot(m, u, precision=_HI,
                       preferred_element_type=jnp.float32)
        rows = jnp.sum(m, axis=1, keepdims=True)
        rexc = jnp.dot(t, rows, precision=_HI,
                       preferred_element_type=jnp.float32)
        return wexc + rexc, jnp.sum(rows)

    p0, c0 = prefix(m0)
    p1, c1 = prefix(m1)
    pos = jnp.where(d < NH, p0, LHALF + p1).astype(jnp.int32)
    pos_out[...] = pos + (pl.program_id(0) % NTILE) * LTILE
    cnt = jnp.concatenate(
        [c0.reshape(1, 1), c1.reshape(1, 1), jnp.zeros((1, 126))], axis=1)
    cnt_out[...] = (cnt + jnp.zeros((8, 128))).astype(jnp.int32)


_tc_pos = pl.pallas_call(
    _tc_pos_body,
    grid=(NW,),
    in_specs=[pl.BlockSpec((WPT, WIN), lambda i: (i, 0))],
    out_specs=[pl.BlockSpec((WPT, WIN), lambda i: (i, 0)),
               pl.BlockSpec((8, 128), lambda i: (i, 0))],
    out_shape=[jax.ShapeDtypeStruct((NW * WPT, WIN), jnp.int32),
               jax.ShapeDtypeStruct((NW * 8, 128), jnp.int32)],
)


def _tc_first_body(h0, h1, x, w, y_out, dinv_out):
    deg = h0[...] + h1[...] + 1.0
    row = pl.program_id(0) * R + lax.broadcasted_iota(jnp.int32, (R, 1), 0)
    dinv = jnp.where(row < N, lax.rsqrt(deg), 0.0)
    y_out[...] = jnp.dot(x[...], w[...], precision=_HI,
                         preferred_element_type=jnp.float32) * dinv
    dinv_out[...] = dinv


def _tc_mid_body(p0, p1, y, dinv, b, w, yn_out):
    h = jnp.maximum(dinv[...] * (p0[...] + p1[...] - y[...]) + b[...], 0.0)
    yn_out[...] = jnp.dot(h, w[...], precision=_HI,
                          preferred_element_type=jnp.float32) * dinv[...]


def _tc_last_body(p0, p1, y, dinv, b, out):
    h = dinv[...] * (p0[...] + p1[...] - y[...]) + b[...]
    m = jnp.max(h, axis=-1, keepdims=True)
    s = jnp.sum(jnp.exp(h - m), axis=-1, keepdims=True)
    out[...] = h - m - jnp.log(s)


_col = pl.BlockSpec((R, 1), lambda i: (i, 0))
_rowblk = pl.BlockSpec((R, D), lambda i: (i, 0))
_wspec = pl.BlockSpec((D, D), lambda i: (0, 0))
_bspec = pl.BlockSpec((1, D), lambda i: (0, 0))
_grid = NP // R

_tc_first = pl.pallas_call(
    _tc_first_body,
    grid=(_grid,),
    in_specs=[_col, _col, _rowblk, _wspec],
    out_specs=[_rowblk, _col],
    out_shape=[jax.ShapeDtypeStruct((NP, D), jnp.float32),
               jax.ShapeDtypeStruct((NP, 1), jnp.float32)],
)

_tc_mid = pl.pallas_call(
    _tc_mid_body,
    grid=(_grid,),
    in_specs=[_rowblk, _rowblk, _rowblk, _col, _bspec, _wspec],
    out_specs=_rowblk,
    out_shape=jax.ShapeDtypeStruct((NP, D), jnp.float32),
)

_tc_last = pl.pallas_call(
    _tc_last_body,
    grid=(_grid,),
    in_specs=[_rowblk, _rowblk, _rowblk, _col, _bspec],
    out_specs=_rowblk,
    out_shape=jax.ShapeDtypeStruct((NP, D), jnp.float32),
)


def kernel(x, edge_index, batch, W1, b1, W2, b2, W3, b3):
    src = edge_index[0].astype(jnp.int32)
    dst = edge_index[1].astype(jnp.int32)
    npad = EP - E
    ar = jnp.arange(npad, dtype=jnp.int32)
    srcf = jnp.concatenate([src, N + (ar % 128)])
    dstf = jnp.concatenate([dst, N + (ar % 8)])
    ldf = jnp.where(dstf >= NH, dstf - NH, dstf)
    dstw = dstf.reshape(NW, WPT, WIN)
    srcw = srcf.reshape(NW, WPT, WIN)
    ldw = ldf.reshape(NW, WPT, WIN)
    xp = jnp.zeros((NP, D), jnp.float32).at[:N].set(x)
    zeros1d = jnp.zeros((NP,), jnp.float32)
    tsrc = N + (jnp.arange(LTILE, dtype=jnp.int32) % 128)
    tld = jnp.zeros((LTILE,), jnp.int32)

    posb, cntb = _tc_pos(dstf.reshape(NW * WPT, WIN))
    posw = posb.reshape(NW, WPT, WIN)
    cnt = cntb.reshape(NW, 8, 128)[:, 0, :16]
    hist, lsrc, lld = _sc_prep(dstw, srcw, ldw, posw, tsrc, tld, zeros1d)
    gl = lsrc.reshape(NW, 2, LWPT, WIN)
    ll = lld.reshape(NW, 2, LWPT, WIN)

    y, dinv = _tc_first(hist[0][:, None], hist[1][:, None], xp, W1)
    p = _sc_scatter(gl, ll, cnt, y)
    y = _tc_mid(p[0], p[1], y, dinv, b1.reshape(1, D), W2)
    p = _sc_scatter(gl, ll, cnt, y)
    y = _tc_mid(p[0], p[1], y, dinv, b2.reshape(1, D), W3)
    p = _sc_scatter(gl, ll, cnt, y)
    out = _tc_last(p[0], p[1], y, dinv, b3.reshape(1, D))
    return out[:N]
